# Initial kernel scaffold; baseline (speedup 1.0000x reference)
import jax
import jax.numpy as jnp
from jax.experimental import pallas as pl


def kernel(x, edge_index, batch, topo, W1, b1, W2, b2, Wl, bl):
    raise NotImplementedError("write your pallas kernel here")


# speedup vs baseline: 7.9284x; 7.9284x over previous
import functools

import jax
import jax.numpy as jnp
from jax import lax
from jax.experimental import pallas as pl
from jax.experimental.pallas import tpu as pltpu
from jax.experimental.pallas import tpu_sc as plsc

N = 10000
E = 320000
D = 128
H = 128
TOPO = 32
C = 10
G = 64

NC = 2
NS = 16
NW = NC * NS
NPAD = 10112
ROWS_PER_SUB = NPAD // NS
CHUNK = 128
EPT = 10240
EPAD = EPT * NW
NCHUNK = EPT // CHUNK
DEGW = 16


def _sc_mesh():
    return plsc.VectorSubcoreMesh(
        core_axis_name="c", subcore_axis_name="s", num_cores=NC, num_subcores=NS
    )


@functools.partial(
    pl.kernel,
    out_type=jax.ShapeDtypeStruct((NC, NPAD, DEGW), jnp.float32),
    mesh=_sc_mesh(),
    scratch_types=[
        pltpu.VMEM((CHUNK,), jnp.int32),
        pltpu.VMEM((CHUNK, DEGW), jnp.float32),
        pltpu.VMEM_SHARED((NPAD, DEGW), jnp.float32),
    ],
)
def _sc_degree(dst_hbm, zrow_hbm, out_hbm, idx_v, ones_v, acc_sh):
    cid = lax.axis_index("c")
    sid = lax.axis_index("s")
    wid = cid * NS + sid

    for i in range(CHUNK):
        ones_v[i, :] = jnp.ones((DEGW,), jnp.float32)

    pltpu.sync_copy(zrow_hbm, acc_sh.at[pl.ds(sid * ROWS_PER_SUB, ROWS_PER_SUB)])
    plsc.subcore_barrier()

    base = wid * EPT

    def body(i, carry):
        pltpu.sync_copy(dst_hbm.at[pl.ds(base + i * CHUNK, CHUNK)], idx_v)
        pltpu.sync_copy(ones_v, acc_sh.at[idx_v], add=True)
        return carry

    lax.fori_loop(0, NCHUNK, body, 0)
    plsc.subcore_barrier()

    pltpu.sync_copy(
        acc_sh.at[pl.ds(sid * ROWS_PER_SUB, ROWS_PER_SUB)],
        out_hbm.at[cid, pl.ds(sid * ROWS_PER_SUB, ROWS_PER_SUB)],
    )


@functools.partial(
    pl.kernel,
    out_type=jax.ShapeDtypeStruct((NC, NPAD, H), jnp.float32),
    mesh=_sc_mesh(),
    scratch_types=[
        pltpu.VMEM((CHUNK,), jnp.int32),
        pltpu.VMEM((CHUNK,), jnp.int32),
        pltpu.VMEM((CHUNK, H), jnp.float32),
        pltpu.SemaphoreType.DMA,
        pltpu.VMEM_SHARED((NPAD, H), jnp.float32),
    ],
)
def _sc_spmm(table_hbm, src_hbm, dst_hbm, zrow_hbm, out_hbm,
             src_v, dst_v, rows_v, sem, acc_sh):
    cid = lax.axis_index("c")
    sid = lax.axis_index("s")
    wid = cid * NS + sid

    pltpu.sync_copy(zrow_hbm, acc_sh.at[pl.ds(sid * ROWS_PER_SUB, ROWS_PER_SUB)])
    plsc.subcore_barrier()

    base = wid * EPT

    def body(i, carry):
        off = base + i * CHUNK
        pltpu.sync_copy(src_hbm.at[pl.ds(off, CHUNK)], src_v)
        pltpu.sync_copy(dst_hbm.at[pl.ds(off, CHUNK)], dst_v)
        pltpu.async_copy(table_hbm.at[src_v], rows_v, sem).wait()
        pltpu.sync_copy(rows_v, acc_sh.at[dst_v], add=True)
        return carry

    lax.fori_loop(0, NCHUNK, body, 0)
    plsc.subcore_barrier()

    pltpu.sync_copy(
        acc_sh.at[pl.ds(sid * ROWS_PER_SUB, ROWS_PER_SUB)],
        out_hbm.at[cid, pl.ds(sid * ROWS_PER_SUB, ROWS_PER_SUB)],
    )


def _tc_prescale_body(x_ref, w_ref, degp_ref, s_ref, d2_ref, dinv_ref):
    deg = degp_ref[0, 0:N, 0:1] + degp_ref[1, 0:N, 0:1] + 1.0
    dinv = lax.rsqrt(deg)
    xw = jnp.dot(x_ref[...], w_ref[...], preferred_element_type=jnp.float32)
    s = dinv * xw
    s_ref[...] = s
    d2_ref[...] = dinv * s
    dinv_ref[...] = dinv


_tc_prescale = pl.pallas_call(
    _tc_prescale_body,
    out_shape=[
        jax.ShapeDtypeStruct((N, H), jnp.float32),
        jax.ShapeDtypeStruct((N, H), jnp.float32),
        jax.ShapeDtypeStruct((N, 1), jnp.float32),
    ],
)


def _tc_mid_body(p_ref, d2_ref, dinv_ref, b1_ref, w2_ref, s_ref, d2o_ref):
    agg = p_ref[0, 0:N, :] + p_ref[1, 0:N, :]
    dinv = dinv_ref[...]
    h1 = jnp.maximum(dinv * agg + d2_ref[...] + b1_ref[...], 0.0)
    xw = jnp.dot(h1, w2_ref[...], preferred_element_type=jnp.float32)
    s = dinv * xw
    s_ref[...] = s
    d2o_ref[...] = dinv * s


_tc_mid = pl.pallas_call(
    _tc_mid_body,
    out_shape=[
        jax.ShapeDtypeStruct((N, H), jnp.float32),
        jax.ShapeDtypeStruct((N, H), jnp.float32),
    ],
)


def _tc_head_body(q_ref, d2_ref, dinv_ref, b2_ref, batch_ref, topo_ref,
                  wl_ref, bl_ref, out_ref):
    agg = q_ref[0, 0:N, :] + q_ref[1, 0:N, :]
    h2 = jnp.maximum(dinv_ref[...] * agg + d2_ref[...] + b2_ref[...], 0.0)
    gid = lax.broadcasted_iota(jnp.int32, (N, G), 1)
    onehot = jnp.where(gid == batch_ref[...], 1.0, 0.0)
    ssum = lax.dot_general(
        onehot, h2, (((0,), (0,)), ((), ())), preferred_element_type=jnp.float32
    )
    cnt = lax.dot_general(
        onehot, jnp.ones((N, 1), jnp.float32), (((0,), (0,)), ((), ())),
        preferred_element_type=jnp.float32,
    )
    g = jnp.where(cnt > 0.0, ssum / jnp.maximum(cnt, 1.0), 0.0)
    gt = jnp.concatenate([g, topo_ref[...]], axis=1)
    out_ref[...] = (
        jnp.dot(gt, wl_ref[...], preferred_element_type=jnp.float32) + bl_ref[...]
    )


_tc_head = pl.pallas_call(
    _tc_head_body,
    out_shape=jax.ShapeDtypeStruct((G, C), jnp.float32),
)


def kernel(x, edge_index, batch, topo, W1, b1, W2, b2, Wl, bl):
    src = edge_index[0]
    dst = edge_index[1]
    pad = EPAD - E
    src_p = jnp.concatenate([src, jnp.zeros((pad,), jnp.int32)])
    dst_p = jnp.concatenate([dst, jnp.full((pad,), N, jnp.int32)])

    zdeg = jnp.zeros((ROWS_PER_SUB, DEGW), jnp.float32)
    zrow = jnp.zeros((ROWS_PER_SUB, H), jnp.float32)

    degp = _sc_degree(dst_p, zdeg)
    s1, d2xw1, dinv = _tc_prescale(x, W1, degp)
    p = _sc_spmm(s1, src_p, dst_p, zrow)
    s2, d2xw2 = _tc_mid(p, d2xw1, dinv, b1.reshape(1, H), W2)
    q = _sc_spmm(s2, src_p, dst_p, zrow)
    out = _tc_head(
        q, d2xw2, dinv, b2.reshape(1, H), batch.reshape(N, 1), topo,
        Wl, bl.reshape(1, C),
    )
    return out


# speedup vs baseline: 10.0385x; 1.2661x over previous
import functools

import jax
import jax.numpy as jnp
from jax import lax
from jax.experimental import pallas as pl
from jax.experimental.pallas import tpu as pltpu
from jax.experimental.pallas import tpu_sc as plsc

N = 10000
E = 320000
D = 128
H = 128
TOPO = 32
C = 10
G = 64

NC = 2
NS = 16
NW = NC * NS
NPAD = 10112
ROWS_PER_SUB = NPAD // NS
CHUNK = 128
EPT = 10240
EPAD = EPT * NW
NCHUNK = EPT // CHUNK
DEGW = 16


def _sc_mesh():
    return plsc.VectorSubcoreMesh(
        core_axis_name="c", subcore_axis_name="s", num_cores=NC, num_subcores=NS
    )


@functools.partial(
    pl.kernel,
    out_type=jax.ShapeDtypeStruct((NC, NPAD, DEGW), jnp.float32),
    mesh=_sc_mesh(),
    scratch_types=[
        pltpu.VMEM((NCHUNK, CHUNK), jnp.int32),
        pltpu.VMEM((CHUNK, DEGW), jnp.float32),
        pltpu.SemaphoreType.DMA,
        pltpu.VMEM_SHARED((NPAD, DEGW), jnp.float32),
    ],
)
def _sc_degree(dst_hbm, zrow_hbm, out_hbm, dst_all, ones_v, sem, acc_sh):
    cid = lax.axis_index("c")
    sid = lax.axis_index("s")
    wid = cid * NS + sid

    for i in range(CHUNK):
        ones_v[i, :] = jnp.ones((DEGW,), jnp.float32)

    pltpu.sync_copy(dst_hbm.at[wid], dst_all)
    pltpu.sync_copy(zrow_hbm, acc_sh.at[pl.ds(sid * ROWS_PER_SUB, ROWS_PER_SUB)])
    plsc.subcore_barrier()

    FIRE = 8

    def body(g, carry):
        for b in range(FIRE):
            pltpu.async_copy(ones_v, acc_sh.at[dst_all.at[g * FIRE + b]], sem,
                             add=True)
        for b in range(FIRE):
            pltpu.make_async_copy(
                ones_v, acc_sh.at[dst_all.at[g * FIRE + b]], sem
            ).wait()
        return carry

    lax.fori_loop(0, NCHUNK // FIRE, body, 0)
    plsc.subcore_barrier()

    pltpu.sync_copy(
        acc_sh.at[pl.ds(sid * ROWS_PER_SUB, ROWS_PER_SUB)],
        out_hbm.at[cid, pl.ds(sid * ROWS_PER_SUB, ROWS_PER_SUB)],
    )


@functools.partial(
    pl.kernel,
    out_type=jax.ShapeDtypeStruct((NC, NPAD, H), jnp.float32),
    mesh=_sc_mesh(),
    scratch_types=[
        pltpu.VMEM((2, CHUNK), jnp.int32),
        pltpu.VMEM((NCHUNK, CHUNK), jnp.int32),
        pltpu.VMEM((2, CHUNK, H), jnp.float32),
        pltpu.SemaphoreType.DMA((2,)),
        pltpu.SemaphoreType.DMA((2,)),
        pltpu.SemaphoreType.DMA((2,)),
        pltpu.VMEM_SHARED((NPAD, H), jnp.float32),
    ],
)
def _sc_spmm(table_hbm, src_hbm, dst_hbm, zrow_hbm, out_hbm,
             si_v, dst_all, rows_v, isem, gsem, ssem, acc_sh):
    cid = lax.axis_index("c")
    sid = lax.axis_index("s")
    wid = cid * NS + sid

    pltpu.sync_copy(dst_hbm.at[wid], dst_all)
    pltpu.sync_copy(zrow_hbm, acc_sh.at[pl.ds(sid * ROWS_PER_SUB, ROWS_PER_SUB)])
    plsc.subcore_barrier()

    def idxload(i, b):
        pltpu.async_copy(src_hbm.at[wid, i], si_v.at[b], isem.at[b])

    def idx_wait(i, b):
        pltpu.make_async_copy(src_hbm.at[wid, i], si_v.at[b], isem.at[b]).wait()

    def gather(b):
        pltpu.async_copy(table_hbm.at[si_v.at[b]], rows_v.at[b], gsem.at[b])

    def gather_wait(b):
        pltpu.make_async_copy(
            table_hbm.at[si_v.at[b]], rows_v.at[b], gsem.at[b]
        ).wait()

    def scatter(i, b):
        pltpu.async_copy(rows_v.at[b], acc_sh.at[dst_all.at[i]], ssem.at[b],
                         add=True)

    def scatter_wait(i, b):
        pltpu.make_async_copy(
            rows_v.at[b], acc_sh.at[dst_all.at[i]], ssem.at[b]
        ).wait()

    idxload(0, 0)
    idxload(1, 1)
    idx_wait(0, 0)
    gather(0)

    def body(g, carry):
        for b in range(2):
            i2 = g * 2 + b
            nb = 1 - b

            @pl.when(i2 >= 1)
            def _():
                scatter_wait(i2 - 1, nb)

            @pl.when(i2 + 1 < NCHUNK)
            def _():
                idx_wait(i2 + 1, nb)
                gather(nb)

            gather_wait(b)
            scatter(i2, b)

            @pl.when(i2 + 2 < NCHUNK)
            def _():
                idxload(i2 + 2, b)
        return carry

    lax.fori_loop(0, NCHUNK // 2, body, 0)
    scatter_wait(NCHUNK - 1, (NCHUNK - 1) % 2)

    plsc.subcore_barrier()

    pltpu.sync_copy(
        acc_sh.at[pl.ds(sid * ROWS_PER_SUB, ROWS_PER_SUB)],
        out_hbm.at[cid, pl.ds(sid * ROWS_PER_SUB, ROWS_PER_SUB)],
    )


def _tc_prescale_body(x_ref, w_ref, degp_ref, s_ref, d2_ref, dinv_ref):
    deg = degp_ref[0, 0:N, 0:1] + degp_ref[1, 0:N, 0:1] + 1.0
    dinv = lax.rsqrt(deg)
    xw = jnp.dot(x_ref[...], w_ref[...], preferred_element_type=jnp.float32)
    s = dinv * xw
    s_ref[...] = s
    d2_ref[...] = dinv * s
    dinv_ref[...] = dinv


_tc_prescale = pl.pallas_call(
    _tc_prescale_body,
    out_shape=[
        jax.ShapeDtypeStruct((N, H), jnp.float32),
        jax.ShapeDtypeStruct((N, H), jnp.float32),
        jax.ShapeDtypeStruct((N, 1), jnp.float32),
    ],
)


def _tc_mid_body(p_ref, d2_ref, dinv_ref, b1_ref, w2_ref, s_ref, d2o_ref):
    agg = p_ref[0, 0:N, :] + p_ref[1, 0:N, :]
    dinv = dinv_ref[...]
    h1 = jnp.maximum(dinv * agg + d2_ref[...] + b1_ref[...], 0.0)
    xw = jnp.dot(h1, w2_ref[...], preferred_element_type=jnp.float32)
    s = dinv * xw
    s_ref[...] = s
    d2o_ref[...] = dinv * s


_tc_mid = pl.pallas_call(
    _tc_mid_body,
    out_shape=[
        jax.ShapeDtypeStruct((N, H), jnp.float32),
        jax.ShapeDtypeStruct((N, H), jnp.float32),
    ],
)


def _tc_head_body(q_ref, d2_ref, dinv_ref, b2_ref, batch_ref, topo_ref,
                  wl_ref, bl_ref, out_ref):
    agg = q_ref[0, 0:N, :] + q_ref[1, 0:N, :]
    h2 = jnp.maximum(dinv_ref[...] * agg + d2_ref[...] + b2_ref[...], 0.0)
    gid = lax.broadcasted_iota(jnp.int32, (N, G), 1)
    onehot = jnp.where(gid == batch_ref[...], 1.0, 0.0)
    ssum = lax.dot_general(
        onehot, h2, (((0,), (0,)), ((), ())), preferred_element_type=jnp.float32
    )
    cnt = lax.dot_general(
        onehot, jnp.ones((N, 1), jnp.float32), (((0,), (0,)), ((), ())),
        preferred_element_type=jnp.float32,
    )
    g = jnp.where(cnt > 0.0, ssum / jnp.maximum(cnt, 1.0), 0.0)
    gt = jnp.concatenate([g, topo_ref[...]], axis=1)
    out_ref[...] = (
        jnp.dot(gt, wl_ref[...], preferred_element_type=jnp.float32) + bl_ref[...]
    )


_tc_head = pl.pallas_call(
    _tc_head_body,
    out_shape=jax.ShapeDtypeStruct((G, C), jnp.float32),
)


def kernel(x, edge_index, batch, topo, W1, b1, W2, b2, Wl, bl):
    src = edge_index[0]
    dst = edge_index[1]
    pad = EPAD - E
    src_p = jnp.concatenate([src, jnp.zeros((pad,), jnp.int32)])
    dst_p = jnp.concatenate([dst, jnp.full((pad,), N, jnp.int32)])
    src3 = src_p.reshape(NW, NCHUNK, CHUNK)
    dst3 = dst_p.reshape(NW, NCHUNK, CHUNK)

    zdeg = jnp.zeros((ROWS_PER_SUB, DEGW), jnp.float32)
    zrow = jnp.zeros((ROWS_PER_SUB, H), jnp.float32)

    degp = _sc_degree(dst3, zdeg)
    s1, d2xw1, dinv = _tc_prescale(x, W1, degp)
    p = _sc_spmm(s1, src3, dst3, zrow)
    s2, d2xw2 = _tc_mid(p, d2xw1, dinv, b1.reshape(1, H), W2)
    q = _sc_spmm(s2, src3, dst3, zrow)
    out = _tc_head(
        q, d2xw2, dinv, b2.reshape(1, H), batch.reshape(N, 1), topo,
        Wl, bl.reshape(1, C),
    )
    return out
